# baseline (speedup 1.0000x reference)
import functools

import jax
import jax.numpy as jnp
from jax import lax
from jax.experimental import pallas as pl
from jax.experimental.pallas import tpu as pltpu
from jax.experimental.pallas import tpu_sc as plsc

N = 10000
NP = 10240
E = 320000
D_IN, D_HID, D_OUT = 128, 64, 2
D_AGG2 = 16
NC, NS = 2, 16
NW = NC * NS
CH = 128
NCH = 80
EPW = NCH * CH
EP = NW * EPW
PAD_NODE = N + 100
RPT = NP // NS
NBUF = 8

_MESH = plsc.VectorSubcoreMesh(
    core_axis_name="c", subcore_axis_name="s", num_cores=NC, num_subcores=NS)
_SC_PARAMS = pltpu.CompilerParams(use_tc_tiling_on_sc=False)


def _deg_body(ei_hbm, ones_hbm, zeros_hbm, deg_out, idx_v, ones_v, dsem, acc):
  c = lax.axis_index("c")
  s = lax.axis_index("s")
  w = c * NS + s
  pltpu.sync_copy(ei_hbm.at[1].at[w], idx_v)
  pltpu.sync_copy(ones_hbm, ones_v)
  pltpu.sync_copy(zeros_hbm.at[pl.ds(s * RPT, RPT)],
                  acc.at[pl.ds(s * RPT, RPT)])
  plsc.subcore_barrier()

  def body(j, carry):
    pltpu.async_copy(ones_v, acc.at[idx_v.at[j]], dsem, add=True)

    @pl.when(j >= NBUF)
    def _():
      pltpu.make_async_copy(ones_v, acc.at[idx_v.at[0]], dsem).wait()

    return carry

  lax.fori_loop(0, NCH, body, 0)
  for _ in range(NBUF):
    pltpu.make_async_copy(ones_v, acc.at[idx_v.at[0]], dsem).wait()
  plsc.subcore_barrier()
  pltpu.sync_copy(acc.at[pl.ds(s * RPT, RPT)],
                  deg_out.at[c].at[pl.ds(s * RPT, RPT)])


_deg_call = functools.partial(
    pl.kernel,
    out_type=jax.ShapeDtypeStruct((NC, NP, 1), jnp.float32),
    mesh=_MESH,
    compiler_params=_SC_PARAMS,
    scratch_types=[
        pltpu.VMEM((NCH, CH), jnp.int32),
        pltpu.VMEM((CH, 1), jnp.float32),
        pltpu.SemaphoreType.DMA,
        pltpu.VMEM_SHARED((NP, 1), jnp.float32),
    ],
)(_deg_body)


def _make_agg(d):
  def _agg_body(hs_hbm, ei_hbm, out_hbm,
                idxs_v, idxd_v, rows, gsems, ssems, acc):
    c = lax.axis_index("c")
    s = lax.axis_index("s")
    w = c * NS + s
    pltpu.sync_copy(ei_hbm.at[0].at[w], idxs_v)
    pltpu.sync_copy(ei_hbm.at[1].at[w], idxd_v)
    pltpu.sync_copy(hs_hbm.at[pl.ds(s * RPT, RPT)],
                    acc.at[pl.ds(s * RPT, RPT)])
    plsc.subcore_barrier()

    half = NBUF // 2
    for b in range(half):
      pltpu.async_copy(hs_hbm.at[idxs_v.at[b]], rows[b], gsems[b])

    def body(t, carry):
      for b in range(NBUF):
        j = NBUF * t + b
        bg = (b + half) % NBUF
        pltpu.make_async_copy(hs_hbm.at[idxs_v.at[j]], rows[b],
                              gsems[b]).wait()
        pltpu.async_copy(rows[b], acc.at[idxd_v.at[j]], ssems[b], add=True)
        @pl.when(j >= half)
        def _():
          pltpu.make_async_copy(rows[bg], acc.at[idxd_v.at[j - half]],
                                ssems[bg]).wait()

        @pl.when(j + half < NCH)
        def _():
          pltpu.async_copy(hs_hbm.at[idxs_v.at[j + half]], rows[bg],
                           gsems[bg])
      return carry

    lax.fori_loop(0, NCH // NBUF, body, 0)
    for b in range(half, NBUF):
      j = NCH - NBUF + b
      pltpu.make_async_copy(rows[b], acc.at[idxd_v.at[j]], ssems[b]).wait()
    plsc.subcore_barrier()
    pltpu.sync_copy(acc.at[pl.ds(s * RPT, RPT)],
                    out_hbm.at[c].at[pl.ds(s * RPT, RPT)])

  return functools.partial(
      pl.kernel,
      out_type=jax.ShapeDtypeStruct((NC, NP, d), jnp.float32),
      mesh=_MESH,
      compiler_params=_SC_PARAMS,
      scratch_types=[
          pltpu.VMEM((NCH, CH), jnp.int32),
          pltpu.VMEM((NCH, CH), jnp.int32),
          [pltpu.VMEM((CH, d), jnp.float32)] * NBUF,
          [pltpu.SemaphoreType.DMA] * NBUF,
          [pltpu.SemaphoreType.DMA] * NBUF,
          pltpu.VMEM_SHARED((NP, d), jnp.float32),
      ],
  )(_agg_body)


_agg64_call = _make_agg(D_HID)
_agg2_call = _make_agg(D_AGG2)


def _mm1_body(x_ref, w1_ref, degp_ref, o_ref):
  dis = lax.rsqrt(degp_ref[0] + degp_ref[1] + 1.0)
  u = jnp.dot(x_ref[...], w1_ref[...], preferred_element_type=jnp.float32)
  o_ref[pl.ds(0, N), :] = u * dis[:N]
  o_ref[pl.ds(N, NP - N), :] = jnp.zeros((NP - N, D_HID), jnp.float32)


_mm1_call = pl.pallas_call(
    _mm1_body,
    out_shape=jax.ShapeDtypeStruct((NP, D_HID), jnp.float32),
)


def _mm2_body(degp_ref, hs1_ref, cp_ref, b1_ref, w2_ref, o_ref):
  dis = lax.rsqrt(degp_ref[0] + degp_ref[1] + 1.0)
  p = cp_ref[0] + cp_ref[1] - hs1_ref[...]
  h1 = jnp.maximum(dis * p + b1_ref[...], 0.0)
  o_ref[...] = jnp.dot(h1, w2_ref[...],
                       preferred_element_type=jnp.float32) * dis


_mm2_call = pl.pallas_call(
    _mm2_body,
    out_shape=jax.ShapeDtypeStruct((NP, D_AGG2), jnp.float32),
)


def _soft_body(degp_ref, hs2_ref, qp_ref, b2_ref, o_ref):
  dis = lax.rsqrt(degp_ref[0] + degp_ref[1] + 1.0)
  z = dis * (qp_ref[0] + qp_ref[1] - hs2_ref[...]) + b2_ref[...]
  z = z[:N, :D_OUT]
  m = jnp.max(z, axis=-1, keepdims=True)
  e = jnp.exp(z - m)
  o_ref[...] = e / jnp.sum(e, axis=-1, keepdims=True)


_soft_call = pl.pallas_call(
    _soft_body,
    out_shape=jax.ShapeDtypeStruct((N, D_OUT), jnp.float32),
)


def kernel(x, edge_index, W1, b1, W2, b2):
  ei = edge_index.astype(jnp.int32)
  ei3 = jnp.pad(ei, ((0, 0), (0, EP - E)),
                constant_values=PAD_NODE).reshape(2, NW, NCH, CH)
  ones_c = jnp.ones((CH, 1), jnp.float32)
  zeros_np = jnp.zeros((NP, 1), jnp.float32)
  w2p = jnp.zeros((D_HID, D_AGG2), jnp.float32).at[:, :D_OUT].set(W2)
  b2p = jnp.zeros((1, D_AGG2), jnp.float32).at[:, :D_OUT].set(b2)

  degp = _deg_call(ei3, ones_c, zeros_np)
  hs1 = _mm1_call(x, W1, degp)
  cp = _agg64_call(hs1, ei3)
  hs2 = _mm2_call(degp, hs1, cp, b1.reshape(1, -1), w2p)
  qp = _agg2_call(hs2, ei3)
  return _soft_call(degp, hs2, qp, b2p)


# speedup vs baseline: 2.1593x; 2.1593x over previous
import functools

import jax
import jax.numpy as jnp
from jax import lax
from jax.experimental import pallas as pl
from jax.experimental.pallas import tpu as pltpu
from jax.experimental.pallas import tpu_sc as plsc

N = 10000
NP = 10240
E = 320000
D_IN, D_HID, D_OUT = 128, 64, 2
D_AGG2 = 16
NC, NS = 2, 16
NW = NC * NS
CH = 128
NCH = 80
EPW = NCH * CH
EP = NW * EPW
_PAD_IDS = None
RPT = NP // NS
NBUF = 8

_MESH = plsc.VectorSubcoreMesh(
    core_axis_name="c", subcore_axis_name="s", num_cores=NC, num_subcores=NS)
_SC_PARAMS = pltpu.CompilerParams(use_tc_tiling_on_sc=False)


def _deg_body(ei_hbm, ones_hbm, zeros_hbm, deg_out, idx_v, ones_v, dsem, acc):
  c = lax.axis_index("c")
  s = lax.axis_index("s")
  w = c * NS + s
  pltpu.sync_copy(ei_hbm.at[1].at[w], idx_v)
  pltpu.sync_copy(ones_hbm, ones_v)
  pltpu.sync_copy(zeros_hbm.at[pl.ds(s * RPT, RPT)],
                  acc.at[pl.ds(s * RPT, RPT)])
  plsc.subcore_barrier()

  def body(j, carry):
    pltpu.async_copy(ones_v, acc.at[idx_v.at[j]], dsem, add=True)

    @pl.when(j >= NBUF)
    def _():
      pltpu.make_async_copy(ones_v, acc.at[idx_v.at[0]], dsem).wait()

    return carry

  lax.fori_loop(0, NCH, body, 0)
  for _ in range(NBUF):
    pltpu.make_async_copy(ones_v, acc.at[idx_v.at[0]], dsem).wait()
  plsc.subcore_barrier()
  pltpu.sync_copy(acc.at[pl.ds(s * RPT, RPT)],
                  deg_out.at[c].at[pl.ds(s * RPT, RPT)])


_deg_call = functools.partial(
    pl.kernel,
    out_type=jax.ShapeDtypeStruct((NC, NP, 1), jnp.float32),
    mesh=_MESH,
    compiler_params=_SC_PARAMS,
    scratch_types=[
        pltpu.VMEM((NCH, CH), jnp.int32),
        pltpu.VMEM((CH, 1), jnp.float32),
        pltpu.SemaphoreType.DMA,
        pltpu.VMEM_SHARED((NP, 1), jnp.float32),
    ],
)(_deg_body)


def _make_agg(d):
  def _agg_body(hs_hbm, ei_hbm, out_hbm,
                idxs_v, idxd_v, rows, gsems, ssems, acc):
    c = lax.axis_index("c")
    s = lax.axis_index("s")
    w = c * NS + s
    pltpu.sync_copy(ei_hbm.at[0].at[w], idxs_v)
    pltpu.sync_copy(ei_hbm.at[1].at[w], idxd_v)
    pltpu.sync_copy(hs_hbm.at[pl.ds(s * RPT, RPT)],
                    acc.at[pl.ds(s * RPT, RPT)])
    plsc.subcore_barrier()

    half = NBUF // 2
    for b in range(half):
      pltpu.async_copy(hs_hbm.at[idxs_v.at[b]], rows[b], gsems[b])

    def body(t, carry):
      for b in range(NBUF):
        j = NBUF * t + b
        bg = (b + half) % NBUF
        pltpu.make_async_copy(hs_hbm.at[idxs_v.at[j]], rows[b],
                              gsems[b]).wait()
        pltpu.async_copy(rows[b], acc.at[idxd_v.at[j]], ssems[b], add=True)
        @pl.when(j >= half)
        def _():
          pltpu.make_async_copy(rows[bg], acc.at[idxd_v.at[j - half]],
                                ssems[bg]).wait()

        @pl.when(j + half < NCH)
        def _():
          pltpu.async_copy(hs_hbm.at[idxs_v.at[j + half]], rows[bg],
                           gsems[bg])
      return carry

    lax.fori_loop(0, NCH // NBUF, body, 0)
    for b in range(half, NBUF):
      j = NCH - NBUF + b
      pltpu.make_async_copy(rows[b], acc.at[idxd_v.at[j]], ssems[b]).wait()
    plsc.subcore_barrier()
    pltpu.sync_copy(acc.at[pl.ds(s * RPT, RPT)],
                    out_hbm.at[c].at[pl.ds(s * RPT, RPT)])

  return functools.partial(
      pl.kernel,
      out_type=jax.ShapeDtypeStruct((NC, NP, d), jnp.float32),
      mesh=_MESH,
      compiler_params=_SC_PARAMS,
      scratch_types=[
          pltpu.VMEM((NCH, CH), jnp.int32),
          pltpu.VMEM((NCH, CH), jnp.int32),
          [pltpu.VMEM((CH, d), jnp.float32)] * NBUF,
          [pltpu.SemaphoreType.DMA] * NBUF,
          [pltpu.SemaphoreType.DMA] * NBUF,
          pltpu.VMEM_SHARED((NP, d), jnp.float32),
      ],
  )(_agg_body)


_agg64_call = _make_agg(D_HID)
_agg2_call = _make_agg(D_AGG2)


def _mm1_body(x_ref, w1_ref, degp_ref, o_ref):
  dis = lax.rsqrt(degp_ref[0] + degp_ref[1] + 1.0)
  u = jnp.dot(x_ref[...], w1_ref[...], preferred_element_type=jnp.float32)
  o_ref[pl.ds(0, N), :] = u * dis[:N]
  o_ref[pl.ds(N, NP - N), :] = jnp.zeros((NP - N, D_HID), jnp.float32)


_mm1_call = pl.pallas_call(
    _mm1_body,
    out_shape=jax.ShapeDtypeStruct((NP, D_HID), jnp.float32),
)


def _mm2_body(degp_ref, hs1_ref, cp_ref, b1_ref, w2_ref, o_ref):
  dis = lax.rsqrt(degp_ref[0] + degp_ref[1] + 1.0)
  p = cp_ref[0] + cp_ref[1] - hs1_ref[...]
  h1 = jnp.maximum(dis * p + b1_ref[...], 0.0)
  o_ref[...] = jnp.dot(h1, w2_ref[...],
                       preferred_element_type=jnp.float32) * dis


_mm2_call = pl.pallas_call(
    _mm2_body,
    out_shape=jax.ShapeDtypeStruct((NP, D_AGG2), jnp.float32),
)


def _soft_body(degp_ref, hs2_ref, qp_ref, b2_ref, o_ref):
  dis = lax.rsqrt(degp_ref[0] + degp_ref[1] + 1.0)
  z = dis * (qp_ref[0] + qp_ref[1] - hs2_ref[...]) + b2_ref[...]
  z = z[:N, :D_OUT]
  m = jnp.max(z, axis=-1, keepdims=True)
  e = jnp.exp(z - m)
  o_ref[...] = e / jnp.sum(e, axis=-1, keepdims=True)


_soft_call = pl.pallas_call(
    _soft_body,
    out_shape=jax.ShapeDtypeStruct((N, D_OUT), jnp.float32),
)


def kernel(x, edge_index, W1, b1, W2, b2):
  ei = edge_index.astype(jnp.int32)
  pad_ids = (jnp.arange(EP - E, dtype=jnp.int32) % (NP - N)) + N
  pad2 = jnp.broadcast_to(pad_ids, (2, EP - E))
  ei3 = jnp.concatenate([ei, pad2], axis=1).reshape(2, NW, NCH, CH)
  ones_c = jnp.ones((CH, 1), jnp.float32)
  zeros_np = jnp.zeros((NP, 1), jnp.float32)
  w2p = jnp.zeros((D_HID, D_AGG2), jnp.float32).at[:, :D_OUT].set(W2)
  b2p = jnp.zeros((1, D_AGG2), jnp.float32).at[:, :D_OUT].set(b2)

  degp = _deg_call(ei3, ones_c, zeros_np)
  hs1 = _mm1_call(x, W1, degp)
  cp = _agg64_call(hs1, ei3)
  hs2 = _mm2_call(degp, hs1, cp, b1.reshape(1, -1), w2p)
  qp = _agg2_call(hs2, ei3)
  return _soft_call(degp, hs2, qp, b2p)


# speedup vs baseline: 2.2492x; 1.0416x over previous
import functools

import jax
import jax.numpy as jnp
from jax import lax
from jax.experimental import pallas as pl
from jax.experimental.pallas import tpu as pltpu
from jax.experimental.pallas import tpu_sc as plsc

N = 10000
NP = 10240
E = 320000
D_IN, D_HID, D_OUT = 128, 64, 2
D_AGG2 = 16
NC, NS = 2, 16
NW = NC * NS
CH = 128
GCH = E // CH
NCH = 80
RPT = NP // NS
NBUF = 8

_MESH = plsc.VectorSubcoreMesh(
    core_axis_name="c", subcore_axis_name="s", num_cores=NC, num_subcores=NS)
_SC_PARAMS = pltpu.CompilerParams(use_tc_tiling_on_sc=False)


def _chunk(w, j):
  return w + NW * j


def _prefetch_idx(ei_hbm, row, w, idx_v, isem):
  def fire(j, carry):
    @pl.when(_chunk(w, j) < GCH)
    def _():
      pltpu.async_copy(ei_hbm.at[row].at[pl.ds(_chunk(w, j) * CH, CH)],
                       idx_v.at[j], isem)
    return carry

  lax.fori_loop(0, NCH, fire, 0)

  def drain(j, carry):
    @pl.when(_chunk(w, j) < GCH)
    def _():
      pltpu.make_async_copy(ei_hbm.at[row].at[pl.ds(0, CH)],
                            idx_v.at[0], isem).wait()
    return carry

  lax.fori_loop(0, NCH, drain, 0)


def _deg_body(ei_hbm, deg_out, idx_v, ones_v, zeros_v, dsem, acc):
  c = lax.axis_index("c")
  s = lax.axis_index("s")
  w = c * NS + s
  for i in range(CH // 16):
    ones_v[pl.ds(16 * i, 16)] = jnp.ones((16,), jnp.float32)
  for i in range(RPT // 16):
    zeros_v[pl.ds(16 * i, 16)] = jnp.zeros((16,), jnp.float32)
  pltpu.sync_copy(zeros_v, acc.at[pl.ds(s * RPT, RPT)])
  _prefetch_idx(ei_hbm, 1, w, idx_v, dsem)
  plsc.subcore_barrier()

  def body(j, carry):
    @pl.when(_chunk(w, j) < GCH)
    def _():
      pltpu.async_copy(ones_v, acc.at[idx_v.at[j]], dsem, add=True)

    @pl.when((j >= NBUF) & (_chunk(w, j - NBUF) < GCH))
    def _():
      pltpu.make_async_copy(ones_v, acc.at[idx_v.at[0]], dsem).wait()

    return carry

  lax.fori_loop(0, NCH, body, 0)
  for b in range(NBUF):
    @pl.when(_chunk(w, NCH - NBUF + b) < GCH)
    def _():
      pltpu.make_async_copy(ones_v, acc.at[idx_v.at[0]], dsem).wait()
  plsc.subcore_barrier()
  pltpu.sync_copy(acc.at[pl.ds(s * RPT, RPT)],
                  deg_out.at[c].at[pl.ds(s * RPT, RPT)])


_deg_call = functools.partial(
    pl.kernel,
    out_type=jax.ShapeDtypeStruct((NC, NP), jnp.float32),
    mesh=_MESH,
    compiler_params=_SC_PARAMS,
    scratch_types=[
        pltpu.VMEM((NCH, CH), jnp.int32),
        pltpu.VMEM((CH,), jnp.float32),
        pltpu.VMEM((RPT,), jnp.float32),
        pltpu.SemaphoreType.DMA,
        pltpu.VMEM_SHARED((NP,), jnp.float32),
    ],
)(_deg_body)


def _make_agg(d):
  def _agg_body(hs_hbm, ei_hbm, out_hbm,
                idxs_v, idxd_v, rows, isem, gsems, ssems, acc):
    c = lax.axis_index("c")
    s = lax.axis_index("s")
    w = c * NS + s
    pltpu.sync_copy(hs_hbm.at[pl.ds(s * RPT, RPT)],
                    acc.at[pl.ds(s * RPT, RPT)])
    _prefetch_idx(ei_hbm, 0, w, idxs_v, isem)
    _prefetch_idx(ei_hbm, 1, w, idxd_v, isem)
    plsc.subcore_barrier()

    half = NBUF // 2
    for b in range(half):
      pltpu.async_copy(hs_hbm.at[idxs_v.at[b]], rows[b], gsems[b])

    def body(t, carry):
      for b in range(NBUF):
        j = NBUF * t + b
        bg = (b + half) % NBUF
        vj = _chunk(w, j) < GCH
        @pl.when(vj)
        def _():
          pltpu.make_async_copy(hs_hbm.at[idxs_v.at[j]], rows[b],
                                gsems[b]).wait()
          pltpu.async_copy(rows[b], acc.at[idxd_v.at[j]], ssems[b], add=True)

        @pl.when((j >= half) & (_chunk(w, j - half) < GCH))
        def _():
          pltpu.make_async_copy(rows[bg], acc.at[idxd_v.at[j - half]],
                                ssems[bg]).wait()

        @pl.when((j + half < NCH) & (_chunk(w, j + half) < GCH))
        def _():
          pltpu.async_copy(hs_hbm.at[idxs_v.at[j + half]], rows[bg],
                           gsems[bg])
      return carry

    lax.fori_loop(0, NCH // NBUF, body, 0)
    for b in range(half, NBUF):
      j = NCH - NBUF + b
      @pl.when(_chunk(w, j) < GCH)
      def _():
        pltpu.make_async_copy(rows[b], acc.at[idxd_v.at[j]], ssems[b]).wait()
    plsc.subcore_barrier()
    pltpu.sync_copy(acc.at[pl.ds(s * RPT, RPT)],
                    out_hbm.at[c].at[pl.ds(s * RPT, RPT)])

  return functools.partial(
      pl.kernel,
      out_type=jax.ShapeDtypeStruct((NC, NP, d), jnp.float32),
      mesh=_MESH,
      compiler_params=_SC_PARAMS,
      scratch_types=[
          pltpu.VMEM((NCH, CH), jnp.int32),
          pltpu.VMEM((NCH, CH), jnp.int32),
          [pltpu.VMEM((CH, d), jnp.float32)] * NBUF,
          pltpu.SemaphoreType.DMA,
          [pltpu.SemaphoreType.DMA] * NBUF,
          [pltpu.SemaphoreType.DMA] * NBUF,
          pltpu.VMEM_SHARED((NP, d), jnp.float32),
      ],
  )(_agg_body)


_agg64_call = _make_agg(D_HID)
_agg2_call = _make_agg(D_AGG2)


def _mm1_body(x_ref, w1_ref, degp_ref, o_ref):
  dis = lax.rsqrt(degp_ref[0] + degp_ref[1] + 1.0)
  u = jnp.dot(x_ref[...], w1_ref[...], preferred_element_type=jnp.float32)
  o_ref[pl.ds(0, N), :] = u * dis[:N]
  o_ref[pl.ds(N, NP - N), :] = jnp.zeros((NP - N, D_HID), jnp.float32)


_mm1_call = pl.pallas_call(
    _mm1_body,
    out_shape=jax.ShapeDtypeStruct((NP, D_HID), jnp.float32),
)


def _mm2_body(degp_ref, hs1_ref, cp_ref, b1_ref, w2_ref, o_ref):
  dis = lax.rsqrt(degp_ref[0] + degp_ref[1] + 1.0)
  p = cp_ref[0] + cp_ref[1] - hs1_ref[...]
  h1 = jnp.maximum(dis * p + b1_ref[...], 0.0)
  o_ref[...] = jnp.dot(h1, w2_ref[...],
                       preferred_element_type=jnp.float32) * dis


_mm2_call = pl.pallas_call(
    _mm2_body,
    out_shape=jax.ShapeDtypeStruct((NP, D_AGG2), jnp.float32),
)


def _soft_body(degp_ref, hs2_ref, qp_ref, b2_ref, o_ref):
  dis = lax.rsqrt(degp_ref[0] + degp_ref[1] + 1.0)
  z = dis * (qp_ref[0] + qp_ref[1] - hs2_ref[...]) + b2_ref[...]
  z = z[:N, :D_OUT]
  m = jnp.max(z, axis=-1, keepdims=True)
  e = jnp.exp(z - m)
  o_ref[...] = e / jnp.sum(e, axis=-1, keepdims=True)


_soft_call = pl.pallas_call(
    _soft_body,
    out_shape=jax.ShapeDtypeStruct((N, D_OUT), jnp.float32),
)


def kernel(x, edge_index, W1, b1, W2, b2):
  ei = edge_index.astype(jnp.int32)
  w2p = jnp.zeros((D_HID, D_AGG2), jnp.float32).at[:, :D_OUT].set(W2)
  b2p = jnp.zeros((1, D_AGG2), jnp.float32).at[:, :D_OUT].set(b2)

  degp = _deg_call(ei)[..., None]
  hs1 = _mm1_call(x, W1, degp)
  cp = _agg64_call(hs1, ei)
  hs2 = _mm2_call(degp, hs1, cp, b1.reshape(1, -1), w2p)
  qp = _agg2_call(hs2, ei)
  return _soft_call(degp, hs2, qp, b2p)
